# Initial kernel scaffold; baseline (speedup 1.0000x reference)
import functools

import jax
import jax.numpy as jnp
from jax import lax
from jax.experimental import pallas as pl
from jax.experimental.pallas import tpu as pltpu
from jax.experimental.pallas import tpu_sc as plsc

N = 10000
E = 320000
D_IN = 128
F1 = 128
F2 = 64
F3 = 32

NC = 2
NS = 16
NW = NC * NS
EB = 80
E_PER_TILE = E // NW
NB = E_PER_TILE // EB
ROWS_PER_TILE = N // NS
ZROWS = 125
DEG_PAD = 10240
DEG_PER_TILE = DEG_PAD // NS

_sc_mesh = plsc.VectorSubcoreMesh(
    core_axis_name="c", subcore_axis_name="s", num_cores=NC, num_subcores=NS
)


@functools.partial(
    pl.kernel,
    out_type=jax.ShapeDtypeStruct((NC, DEG_PAD), jnp.float32),
    mesh=_sc_mesh,
    scratch_types=[
        pltpu.VMEM((EB,), jnp.int32),
        pltpu.VMEM((EB,), jnp.float32),
        pltpu.VMEM((DEG_PER_TILE,), jnp.float32),
        pltpu.VMEM_SHARED((DEG_PAD,), jnp.float32),
    ],
)
def _deg_kernel(dst_hbm, out_hbm, idx_v, ones_v, zeros_v, deg_sh):
    c = lax.axis_index("c")
    s = lax.axis_index("s")
    wid = s * NC + c
    for k in range(EB // 16):
        ones_v[pl.ds(k * 16, 16)] = jnp.ones((16,), jnp.float32)

    def _zfill(i, carry):
        zeros_v[pl.ds(i * 16, 16)] = jnp.zeros((16,), jnp.float32)
        return carry

    lax.fori_loop(0, DEG_PER_TILE // 16, _zfill, 0)
    pltpu.sync_copy(zeros_v, deg_sh.at[pl.ds(s * DEG_PER_TILE, DEG_PER_TILE)])
    plsc.subcore_barrier()

    base = wid * E_PER_TILE

    def _body(j, carry):
        pltpu.sync_copy(dst_hbm.at[pl.ds(base + j * EB, EB)], idx_v)
        pltpu.sync_copy(ones_v, deg_sh.at[idx_v], add=True)
        return carry

    lax.fori_loop(0, NB, _body, 0)
    plsc.subcore_barrier()
    pltpu.sync_copy(
        deg_sh.at[pl.ds(s * DEG_PER_TILE, DEG_PER_TILE)],
        out_hbm.at[c, pl.ds(s * DEG_PER_TILE, DEG_PER_TILE)],
    )


def _make_spmm(d):
    @functools.partial(
        pl.kernel,
        out_type=jax.ShapeDtypeStruct((NC, N, d), jnp.float32),
        mesh=_sc_mesh,
        scratch_types=[
            pltpu.VMEM((EB,), jnp.int32),
            pltpu.VMEM((EB,), jnp.int32),
            pltpu.VMEM((EB, d), jnp.float32),
            pltpu.VMEM((ZROWS, d), jnp.float32),
            pltpu.VMEM_SHARED((N, d), jnp.float32),
            pltpu.SemaphoreType.DMA,
        ],
    )
    def _spmm(src_hbm, dst_hbm, g_hbm, out_hbm, si_v, di_v, rows_v, z_v, y_sh, sem):
        c = lax.axis_index("c")
        s = lax.axis_index("s")
        wid = s * NC + c

        def _zfill(i, carry):
            for k in range(d // 16):
                z_v[i, pl.ds(k * 16, 16)] = jnp.zeros((16,), jnp.float32)
            return carry

        lax.fori_loop(0, ZROWS, _zfill, 0)
        r0 = s * ROWS_PER_TILE
        for t in range(ROWS_PER_TILE // ZROWS):
            pltpu.sync_copy(z_v, y_sh.at[pl.ds(r0 + t * ZROWS, ZROWS)])
        plsc.subcore_barrier()

        base = wid * E_PER_TILE

        def _body(j, carry):
            e0 = base + j * EB
            pltpu.sync_copy(src_hbm.at[pl.ds(e0, EB)], si_v)
            pltpu.sync_copy(dst_hbm.at[pl.ds(e0, EB)], di_v)
            pltpu.async_copy(g_hbm.at[si_v], rows_v, sem).wait()
            pltpu.sync_copy(rows_v, y_sh.at[di_v], add=True)
            return carry

        lax.fori_loop(0, NB, _body, 0)
        plsc.subcore_barrier()
        pltpu.sync_copy(
            y_sh.at[pl.ds(r0, ROWS_PER_TILE)],
            out_hbm.at[c, pl.ds(r0, ROWS_PER_TILE)],
        )

    return _spmm


_spmm_128 = _make_spmm(F1)
_spmm_64 = _make_spmm(F2)
_spmm_32 = _make_spmm(F3)


BN = 500


def _tc_first_body(x_ref, w_ref, d0_ref, d1_ref, g_ref, dinv_ref):
    deg = d0_ref[...] + d1_ref[...] + 1.0
    dinv = lax.rsqrt(deg)
    h = jnp.dot(x_ref[...], w_ref[...], preferred_element_type=jnp.float32)
    g_ref[...] = h * dinv
    dinv_ref[...] = dinv


def _tc_mid_body(s0_ref, s1_ref, g_ref, dinv_ref, b_ref, w_ref, out_ref):
    dinv = dinv_ref[...]
    agg = s0_ref[...] + s1_ref[...] + g_ref[...]
    h = jnp.maximum(agg * dinv + b_ref[...], 0.0)
    out_ref[...] = jnp.dot(h, w_ref[...], preferred_element_type=jnp.float32) * dinv


def _tc_last_body(s0_ref, s1_ref, g_ref, dinv_ref, b_ref, out_ref):
    agg = s0_ref[...] + s1_ref[...] + g_ref[...]
    out_ref[...] = agg * dinv_ref[...] + b_ref[...]


def _row_spec(d):
    return pl.BlockSpec((BN, d), lambda i: (i, 0))


def _full_spec(r, c):
    return pl.BlockSpec((r, c), lambda i: (0, 0))


def _tc_first(x, w, d0, d1, dw):
    return pl.pallas_call(
        _tc_first_body,
        grid=(N // BN,),
        in_specs=[_row_spec(D_IN), _full_spec(D_IN, dw), _row_spec(1), _row_spec(1)],
        out_specs=[_row_spec(dw), _row_spec(1)],
        out_shape=[
            jax.ShapeDtypeStruct((N, dw), jnp.float32),
            jax.ShapeDtypeStruct((N, 1), jnp.float32),
        ],
    )(x, w, d0, d1)


def _tc_mid(s0, s1, g, dinv, b, w, din, dout):
    return pl.pallas_call(
        _tc_mid_body,
        grid=(N // BN,),
        in_specs=[
            _row_spec(din),
            _row_spec(din),
            _row_spec(din),
            _row_spec(1),
            _full_spec(1, din),
            _full_spec(din, dout),
        ],
        out_specs=_row_spec(dout),
        out_shape=jax.ShapeDtypeStruct((N, dout), jnp.float32),
    )(s0, s1, g, dinv, b, w)


def _tc_last(s0, s1, g, dinv, b, d):
    return pl.pallas_call(
        _tc_last_body,
        grid=(N // BN,),
        in_specs=[
            _row_spec(d),
            _row_spec(d),
            _row_spec(d),
            _row_spec(1),
            _full_spec(1, d),
        ],
        out_specs=_row_spec(d),
        out_shape=jax.ShapeDtypeStruct((N, d), jnp.float32),
    )(s0, s1, g, dinv, b)


def kernel(x, edge_index, W1, b1, W2, b2, W3, b3):
    src = edge_index[0]
    dst = edge_index[1]

    degp = _deg_kernel(dst)
    d0 = degp[0, :N, None]
    d1 = degp[1, :N, None]

    g1, dinv = _tc_first(x, W1, d0, d1, F1)
    s1 = _spmm_128(src, dst, g1)
    g2 = _tc_mid(s1[0], s1[1], g1, dinv, b1[None, :], W2, F1, F2)
    s2 = _spmm_64(src, dst, g2)
    g3 = _tc_mid(s2[0], s2[1], g2, dinv, b2[None, :], W3, F2, F3)
    s3 = _spmm_32(src, dst, g3)
    return _tc_last(s3[0], s3[1], g3, dinv, b3[None, :], F3)


# speedup vs baseline: 11.9038x; 11.9038x over previous
import functools

import jax
import jax.numpy as jnp
from jax import lax
from jax.experimental import pallas as pl
from jax.experimental.pallas import tpu as pltpu
from jax.experimental.pallas import tpu_sc as plsc

N = 10000
E = 320000
D_IN = 128
F1 = 128
F2 = 64
F3 = 32

NC = 2
NS = 16
NW = NC * NS
EB = 80
E_PER_TILE = E // NW
NB = E_PER_TILE // EB
N_PAD = 10240
ROWS_PER_TILE = N_PAD // NS
ZROWS = 128
DEG_PAD = 10240
DEG_PER_TILE = DEG_PAD // NS

_sc_mesh = plsc.VectorSubcoreMesh(
    core_axis_name="c", subcore_axis_name="s", num_cores=NC, num_subcores=NS
)


@functools.partial(
    pl.kernel,
    out_type=jax.ShapeDtypeStruct((NC, DEG_PAD), jnp.float32),
    mesh=_sc_mesh,
    scratch_types=[
        pltpu.VMEM((EB,), jnp.int32),
        pltpu.VMEM((EB,), jnp.float32),
        pltpu.VMEM((DEG_PER_TILE,), jnp.float32),
        pltpu.VMEM_SHARED((DEG_PAD,), jnp.float32),
    ],
)
def _deg_kernel(dst_hbm, out_hbm, idx_v, ones_v, zeros_v, deg_sh):
    c = lax.axis_index("c")
    s = lax.axis_index("s")
    wid = s * NC + c
    for k in range(EB // 16):
        ones_v[pl.ds(k * 16, 16)] = jnp.ones((16,), jnp.float32)

    def _zfill(i, carry):
        zeros_v[pl.ds(i * 16, 16)] = jnp.zeros((16,), jnp.float32)
        return carry

    lax.fori_loop(0, DEG_PER_TILE // 16, _zfill, 0)
    pltpu.sync_copy(zeros_v, deg_sh.at[pl.ds(s * DEG_PER_TILE, DEG_PER_TILE)])
    plsc.subcore_barrier()

    base = wid * E_PER_TILE

    def _body(j, carry):
        pltpu.sync_copy(dst_hbm.at[pl.ds(base + j * EB, EB)], idx_v)
        pltpu.sync_copy(ones_v, deg_sh.at[idx_v], add=True)
        return carry

    lax.fori_loop(0, NB, _body, 0)
    plsc.subcore_barrier()
    pltpu.sync_copy(
        deg_sh.at[pl.ds(s * DEG_PER_TILE, DEG_PER_TILE)],
        out_hbm.at[c, pl.ds(s * DEG_PER_TILE, DEG_PER_TILE)],
    )


def _make_spmm(d):
    @functools.partial(
        pl.kernel,
        out_type=jax.ShapeDtypeStruct((NC, N_PAD, d), jnp.float32),
        mesh=_sc_mesh,
        scratch_types=[
            pltpu.VMEM((EB,), jnp.int32),
            pltpu.VMEM((EB,), jnp.int32),
            pltpu.VMEM((EB, d), jnp.float32),
            pltpu.VMEM((ZROWS, d), jnp.float32),
            pltpu.VMEM_SHARED((N_PAD, d), jnp.float32),
            pltpu.SemaphoreType.DMA,
        ],
        compiler_params=pltpu.CompilerParams(use_tc_tiling_on_sc=False),
    )
    def _spmm(src_hbm, dst_hbm, g_hbm, out_hbm, si_v, di_v, rows_v, z_v, y_sh, sem):
        c = lax.axis_index("c")
        s = lax.axis_index("s")
        wid = s * NC + c

        def _zfill(i, carry):
            for k in range(d // 16):
                z_v[i, pl.ds(k * 16, 16)] = jnp.zeros((16,), jnp.float32)
            return carry

        lax.fori_loop(0, ZROWS, _zfill, 0)
        r0 = s * ROWS_PER_TILE
        for t in range(ROWS_PER_TILE // ZROWS):
            pltpu.sync_copy(z_v, y_sh.at[pl.ds(r0 + t * ZROWS, ZROWS)])
        plsc.subcore_barrier()

        base = wid * E_PER_TILE

        def _body(j, carry):
            e0 = base + j * EB
            pltpu.sync_copy(src_hbm.at[pl.ds(e0, EB)], si_v)
            pltpu.sync_copy(dst_hbm.at[pl.ds(e0, EB)], di_v)
            pltpu.async_copy(g_hbm.at[si_v], rows_v, sem).wait()
            pltpu.sync_copy(rows_v, y_sh.at[di_v], add=True)
            return carry

        lax.fori_loop(0, NB, _body, 0)
        plsc.subcore_barrier()
        pltpu.sync_copy(
            y_sh.at[pl.ds(r0, ROWS_PER_TILE)],
            out_hbm.at[c, pl.ds(r0, ROWS_PER_TILE)],
        )

    return _spmm


_spmm_128 = _make_spmm(F1)
_spmm_64 = _make_spmm(F2)
_spmm_32 = _make_spmm(F3)


BN = 400


def _tc_first_body(x_ref, w_ref, d0_ref, d1_ref, g_ref, dinv_ref):
    deg = d0_ref[...] + d1_ref[...] + 1.0
    dinv = lax.rsqrt(deg)
    h = jnp.dot(x_ref[...], w_ref[...], preferred_element_type=jnp.float32)
    g_ref[...] = h * dinv
    dinv_ref[...] = dinv


def _tc_mid_body(s0_ref, s1_ref, g_ref, dinv_ref, b_ref, w_ref, out_ref):
    dinv = dinv_ref[...]
    agg = s0_ref[...] + s1_ref[...] + g_ref[...]
    h = jnp.maximum(agg * dinv + b_ref[...], 0.0)
    out_ref[...] = jnp.dot(h, w_ref[...], preferred_element_type=jnp.float32) * dinv


def _tc_last_body(s0_ref, s1_ref, g_ref, dinv_ref, b_ref, out_ref):
    agg = s0_ref[...] + s1_ref[...] + g_ref[...]
    out_ref[...] = agg * dinv_ref[...] + b_ref[...]


def _row_spec(d):
    return pl.BlockSpec((BN, d), lambda i: (i, 0))


def _full_spec(r, c):
    return pl.BlockSpec((r, c), lambda i: (0, 0))


def _tc_first(x, w, d0, d1, dw):
    return pl.pallas_call(
        _tc_first_body,
        grid=(N // BN,),
        in_specs=[_row_spec(D_IN), _full_spec(D_IN, dw), _row_spec(1), _row_spec(1)],
        out_specs=[_row_spec(dw), _row_spec(1)],
        out_shape=[
            jax.ShapeDtypeStruct((N, dw), jnp.float32),
            jax.ShapeDtypeStruct((N, 1), jnp.float32),
        ],
    )(x, w, d0, d1)


def _tc_mid(s0, s1, g, dinv, b, w, din, dout):
    return pl.pallas_call(
        _tc_mid_body,
        grid=(N // BN,),
        in_specs=[
            _row_spec(din),
            _row_spec(din),
            _row_spec(din),
            _row_spec(1),
            _full_spec(1, din),
            _full_spec(din, dout),
        ],
        out_specs=_row_spec(dout),
        out_shape=jax.ShapeDtypeStruct((N, dout), jnp.float32),
    )(s0, s1, g, dinv, b, w)


def _tc_last(s0, s1, g, dinv, b, d):
    return pl.pallas_call(
        _tc_last_body,
        grid=(N // BN,),
        in_specs=[
            _row_spec(d),
            _row_spec(d),
            _row_spec(d),
            _row_spec(1),
            _full_spec(1, d),
        ],
        out_specs=_row_spec(d),
        out_shape=jax.ShapeDtypeStruct((N, d), jnp.float32),
    )(s0, s1, g, dinv, b)


def kernel(x, edge_index, W1, b1, W2, b2, W3, b3):
    src = edge_index[0]
    dst = edge_index[1]

    degp = _deg_kernel(dst)
    d0 = degp[0, :N, None]
    d1 = degp[1, :N, None]

    g1, dinv = _tc_first(x, W1, d0, d1, F1)
    s1 = _spmm_128(src, dst, g1)
    g2 = _tc_mid(s1[0, :N], s1[1, :N], g1, dinv, b1[None, :], W2, F1, F2)
    s2 = _spmm_64(src, dst, g2)
    g3 = _tc_mid(s2[0, :N], s2[1, :N], g2, dinv, b2[None, :], W3, F2, F3)
    s3 = _spmm_32(src, dst, g3)
    return _tc_last(s3[0, :N], s3[1, :N], g3, dinv, b3[None, :], F3)


# speedup vs baseline: 27.6901x; 2.3261x over previous
import functools

import jax
import jax.numpy as jnp
from jax import lax
from jax.experimental import pallas as pl
from jax.experimental.pallas import tpu as pltpu
from jax.experimental.pallas import tpu_sc as plsc

N = 10000
E = 320000
D_IN = 128
F1 = 128
F2 = 64
F3 = 32

NC = 2
NS = 16
NW = NC * NS
EB = 80
E_PER_TILE = E // NW
NB = E_PER_TILE // EB
N_PAD = 10240
ROWS_PER_TILE = N_PAD // NS
ZROWS = 128
DEG_PAD = 10240
DEG_PER_TILE = DEG_PAD // NS

_sc_mesh = plsc.VectorSubcoreMesh(
    core_axis_name="c", subcore_axis_name="s", num_cores=NC, num_subcores=NS
)


NBUF = 4
NGRP = NB // NBUF - 1


@functools.partial(
    pl.kernel,
    out_type=jax.ShapeDtypeStruct((NC, DEG_PAD), jnp.float32),
    mesh=_sc_mesh,
    scratch_types=[
        pltpu.VMEM((NB, EB), jnp.int32),
        pltpu.VMEM((EB,), jnp.float32),
        pltpu.VMEM((DEG_PER_TILE,), jnp.float32),
        pltpu.VMEM_SHARED((DEG_PAD,), jnp.float32),
        pltpu.SemaphoreType.DMA,
        pltpu.SemaphoreType.DMA,
        pltpu.SemaphoreType.DMA,
        pltpu.SemaphoreType.DMA,
    ],
)
def _deg_kernel(dst_hbm, out_hbm, di_v, ones_v, zeros_v, deg_sh, s0, s1, s2, s3):
    c = lax.axis_index("c")
    s = lax.axis_index("s")
    wid = s * NC + c
    sems = (s0, s1, s2, s3)
    for k in range(EB // 16):
        ones_v[pl.ds(k * 16, 16)] = jnp.ones((16,), jnp.float32)

    def _zfill(i, carry):
        zeros_v[pl.ds(i * 16, 16)] = jnp.zeros((16,), jnp.float32)
        return carry

    lax.fori_loop(0, DEG_PER_TILE // 16, _zfill, 0)
    pltpu.sync_copy(zeros_v, deg_sh.at[pl.ds(s * DEG_PER_TILE, DEG_PER_TILE)])
    pltpu.sync_copy(dst_hbm.at[wid], di_v)
    plsc.subcore_barrier()

    def _scat(j, sem):
        pltpu.async_copy(ones_v, deg_sh.at[di_v.at[j]], sem, add=True)

    def _wait(sem):
        pltpu.make_async_copy(ones_v, deg_sh.at[di_v.at[0]], sem).wait()

    for b in range(NBUF):
        _scat(b, sems[b])

    def _group(g, carry):
        j0 = g * NBUF
        for b in range(NBUF):
            _wait(sems[b])
            _scat(j0 + NBUF + b, sems[b])
        return carry

    lax.fori_loop(0, NGRP, _group, 0)
    for b in range(NBUF):
        _wait(sems[b])
    pltpu.sync_copy(ones_v, deg_sh.at[di_v.at[NB - 1]], add=True)
    plsc.subcore_barrier()
    pltpu.sync_copy(
        deg_sh.at[pl.ds(s * DEG_PER_TILE, DEG_PER_TILE)],
        out_hbm.at[c, pl.ds(s * DEG_PER_TILE, DEG_PER_TILE)],
    )


def _make_spmm(d, nbuf):
    ngrp = NB // nbuf - 1
    tail = NB - (ngrp + 1) * nbuf

    @functools.partial(
        pl.kernel,
        out_type=jax.ShapeDtypeStruct((NC, N_PAD, d), jnp.float32),
        mesh=_sc_mesh,
        scratch_types=[
            pltpu.VMEM((NB, EB), jnp.int32),
            pltpu.VMEM((NB, EB), jnp.int32),
            pltpu.VMEM((nbuf, EB, d), jnp.float32),
            pltpu.VMEM_SHARED((N_PAD, d), jnp.float32),
            [pltpu.SemaphoreType.DMA] * nbuf,
            [pltpu.SemaphoreType.DMA] * nbuf,
        ],
        compiler_params=pltpu.CompilerParams(use_tc_tiling_on_sc=False),
    )
    def _spmm(src_hbm, dst_hbm, g_hbm, out_hbm, si_v, di_v, rows_v, y_sh,
              semg, sems):
        c = lax.axis_index("c")
        s = lax.axis_index("s")
        wid = s * NC + c

        def _zfill(i, carry):
            for b in range(nbuf):
                for k in range(d // 16):
                    rows_v[b, i, pl.ds(k * 16, 16)] = jnp.zeros((16,), jnp.float32)
            return carry

        lax.fori_loop(0, EB, _zfill, 0)
        r0 = s * ROWS_PER_TILE
        nz = ROWS_PER_TILE // EB
        for u in range(nz):
            pltpu.async_copy(
                rows_v.at[u % nbuf], y_sh.at[pl.ds(r0 + u * EB, EB)], semg[0]
            )
        pltpu.sync_copy(src_hbm.at[wid], si_v)
        pltpu.sync_copy(dst_hbm.at[wid], di_v)
        for u in range(nz):
            pltpu.make_async_copy(
                rows_v.at[0], y_sh.at[pl.ds(r0, EB)], semg[0]
            ).wait()
        plsc.subcore_barrier()

        def _gather(j, b):
            pltpu.async_copy(g_hbm.at[si_v.at[j]], rows_v.at[b], semg[b])

        def _gwait(b):
            pltpu.make_async_copy(g_hbm.at[si_v.at[0]], rows_v.at[b], semg[b]).wait()

        def _scat(j, b):
            pltpu.async_copy(rows_v.at[b], y_sh.at[di_v.at[j]], sems[b], add=True)

        def _swait(b):
            pltpu.make_async_copy(rows_v.at[b], y_sh.at[di_v.at[0]], sems[b]).wait()

        for b in range(nbuf):
            _gather(b, b)

        def _group(g, carry):
            j0 = g * nbuf
            for b in range(nbuf):
                _gwait(b)
                _scat(j0 + b, b)
            for b in range(nbuf):
                _swait(b)
                _gather(j0 + nbuf + b, b)
            return carry

        lax.fori_loop(0, ngrp, _group, 0)
        j0 = ngrp * nbuf
        for b in range(nbuf):
            _gwait(b)
            _scat(j0 + b, b)
        for t in range(tail):
            j = (ngrp + 1) * nbuf + t
            _swait(0)
            _gather(j, 0)
            _gwait(0)
            _scat(j, 0)
        _swait(0)
        for b in range(1, nbuf):
            _swait(b)
        plsc.subcore_barrier()
        pltpu.sync_copy(
            y_sh.at[pl.ds(r0, ROWS_PER_TILE)],
            out_hbm.at[c, pl.ds(r0, ROWS_PER_TILE)],
        )

    return _spmm


_spmm_128 = _make_spmm(F1, 2)
_spmm_64 = _make_spmm(F2, 4)
_spmm_32 = _make_spmm(F3, 4)


BN = 400


def _tc_first_body(x_ref, w_ref, d0_ref, d1_ref, g_ref, dinv_ref):
    deg = d0_ref[...] + d1_ref[...] + 1.0
    dinv = lax.rsqrt(deg)
    h = jnp.dot(x_ref[...], w_ref[...], preferred_element_type=jnp.float32)
    g_ref[...] = h * dinv
    dinv_ref[...] = dinv


def _tc_mid_body(s0_ref, s1_ref, g_ref, dinv_ref, b_ref, w_ref, out_ref):
    dinv = dinv_ref[...]
    agg = s0_ref[...] + s1_ref[...] + g_ref[...]
    h = jnp.maximum(agg * dinv + b_ref[...], 0.0)
    out_ref[...] = jnp.dot(h, w_ref[...], preferred_element_type=jnp.float32) * dinv


def _tc_last_body(s0_ref, s1_ref, g_ref, dinv_ref, b_ref, out_ref):
    agg = s0_ref[...] + s1_ref[...] + g_ref[...]
    out_ref[...] = agg * dinv_ref[...] + b_ref[...]


def _row_spec(d):
    return pl.BlockSpec((BN, d), lambda i: (i, 0))


def _full_spec(r, c):
    return pl.BlockSpec((r, c), lambda i: (0, 0))


def _tc_first(x, w, d0, d1, dw):
    return pl.pallas_call(
        _tc_first_body,
        grid=(N // BN,),
        in_specs=[_row_spec(D_IN), _full_spec(D_IN, dw), _row_spec(1), _row_spec(1)],
        out_specs=[_row_spec(dw), _row_spec(1)],
        out_shape=[
            jax.ShapeDtypeStruct((N, dw), jnp.float32),
            jax.ShapeDtypeStruct((N, 1), jnp.float32),
        ],
    )(x, w, d0, d1)


def _tc_mid(s0, s1, g, dinv, b, w, din, dout):
    return pl.pallas_call(
        _tc_mid_body,
        grid=(N // BN,),
        in_specs=[
            _row_spec(din),
            _row_spec(din),
            _row_spec(din),
            _row_spec(1),
            _full_spec(1, din),
            _full_spec(din, dout),
        ],
        out_specs=_row_spec(dout),
        out_shape=jax.ShapeDtypeStruct((N, dout), jnp.float32),
    )(s0, s1, g, dinv, b, w)


def _tc_last(s0, s1, g, dinv, b, d):
    return pl.pallas_call(
        _tc_last_body,
        grid=(N // BN,),
        in_specs=[
            _row_spec(d),
            _row_spec(d),
            _row_spec(d),
            _row_spec(1),
            _full_spec(1, d),
        ],
        out_specs=_row_spec(d),
        out_shape=jax.ShapeDtypeStruct((N, d), jnp.float32),
    )(s0, s1, g, dinv, b)


def kernel(x, edge_index, W1, b1, W2, b2, W3, b3):
    srcr = edge_index[0].reshape(NW, NB, EB)
    dstr = edge_index[1].reshape(NW, NB, EB)

    degp = _deg_kernel(dstr)
    d0 = degp[0, :N, None]
    d1 = degp[1, :N, None]

    g1, dinv = _tc_first(x, W1, d0, d1, F1)
    s1 = _spmm_128(srcr, dstr, g1)
    g2 = _tc_mid(s1[0, :N], s1[1, :N], g1, dinv, b1[None, :], W2, F1, F2)
    s2 = _spmm_64(srcr, dstr, g2)
    g3 = _tc_mid(s2[0, :N], s2[1, :N], g2, dinv, b2[None, :], W3, F2, F3)
    s3 = _spmm_32(srcr, dstr, g3)
    return _tc_last(s3[0, :N], s3[1, :N], g3, dinv, b3[None, :], F3)


# speedup vs baseline: 29.6576x; 1.0711x over previous
import functools

import jax
import jax.numpy as jnp
from jax import lax
from jax.experimental import pallas as pl
from jax.experimental.pallas import tpu as pltpu
from jax.experimental.pallas import tpu_sc as plsc

N = 10000
E = 320000
D_IN = 128
F1 = 128
F2 = 64
F3 = 32

NC = 2
NS = 16
NW = NC * NS
EB = 80
E_PER_TILE = E // NW
NB = E_PER_TILE // EB
N_PAD = 10240
ROWS_PER_TILE = N_PAD // NS
ZROWS = 128
DEG_PAD = 10240
DEG_PER_TILE = DEG_PAD // NS

_sc_mesh = plsc.VectorSubcoreMesh(
    core_axis_name="c", subcore_axis_name="s", num_cores=NC, num_subcores=NS
)


NBUF = 4
NGRP = NB // NBUF - 1


@functools.partial(
    pl.kernel,
    out_type=jax.ShapeDtypeStruct((NC, DEG_PAD), jnp.float32),
    mesh=_sc_mesh,
    scratch_types=[
        pltpu.VMEM((NB, EB), jnp.int32),
        pltpu.VMEM((EB,), jnp.float32),
        pltpu.VMEM((DEG_PER_TILE,), jnp.float32),
        pltpu.VMEM_SHARED((DEG_PAD,), jnp.float32),
        pltpu.SemaphoreType.DMA,
        pltpu.SemaphoreType.DMA,
        pltpu.SemaphoreType.DMA,
        pltpu.SemaphoreType.DMA,
    ],
)
def _deg_kernel(dst_hbm, out_hbm, di_v, ones_v, zeros_v, deg_sh, s0, s1, s2, s3):
    c = lax.axis_index("c")
    s = lax.axis_index("s")
    wid = s * NC + c
    sems = (s0, s1, s2, s3)
    for k in range(EB // 16):
        ones_v[pl.ds(k * 16, 16)] = jnp.ones((16,), jnp.float32)

    def _zfill(i, carry):
        zeros_v[pl.ds(i * 16, 16)] = jnp.zeros((16,), jnp.float32)
        return carry

    lax.fori_loop(0, DEG_PER_TILE // 16, _zfill, 0)
    pltpu.sync_copy(zeros_v, deg_sh.at[pl.ds(s * DEG_PER_TILE, DEG_PER_TILE)])
    pltpu.sync_copy(dst_hbm.at[wid], di_v)
    plsc.subcore_barrier()

    def _scat(j, sem):
        pltpu.async_copy(ones_v, deg_sh.at[di_v.at[j]], sem, add=True)

    def _wait(sem):
        pltpu.make_async_copy(ones_v, deg_sh.at[di_v.at[0]], sem).wait()

    for b in range(NBUF):
        _scat(b, sems[b])

    def _group(g, carry):
        j0 = g * NBUF
        for b in range(NBUF):
            _wait(sems[b])
            _scat(j0 + NBUF + b, sems[b])
        return carry

    lax.fori_loop(0, NGRP, _group, 0)
    for b in range(NBUF):
        _wait(sems[b])
    pltpu.sync_copy(ones_v, deg_sh.at[di_v.at[NB - 1]], add=True)
    plsc.subcore_barrier()
    pltpu.sync_copy(
        deg_sh.at[pl.ds(s * DEG_PER_TILE, DEG_PER_TILE)],
        out_hbm.at[c, pl.ds(s * DEG_PER_TILE, DEG_PER_TILE)],
    )


def _make_spmm(d, nbuf):
    ngrp = NB // nbuf - 1
    tail = NB - (ngrp + 1) * nbuf

    @functools.partial(
        pl.kernel,
        out_type=jax.ShapeDtypeStruct((NC, N_PAD, d), jnp.float32),
        mesh=_sc_mesh,
        scratch_types=[
            pltpu.VMEM((NB, EB), jnp.int32),
            pltpu.VMEM((NB, EB), jnp.int32),
            pltpu.VMEM((nbuf, EB, d), jnp.float32),
            pltpu.VMEM_SHARED((N_PAD, d), jnp.float32),
            [pltpu.SemaphoreType.DMA] * nbuf,
            [pltpu.SemaphoreType.DMA] * nbuf,
        ],
        compiler_params=pltpu.CompilerParams(use_tc_tiling_on_sc=False),
    )
    def _spmm(src_hbm, dst_hbm, g_hbm, out_hbm, si_v, di_v, rows_v, y_sh,
              semg, sems):
        c = lax.axis_index("c")
        s = lax.axis_index("s")
        wid = s * NC + c

        def _zfill(i, carry):
            for b in range(nbuf):
                for k in range(d // 16):
                    rows_v[b, i, pl.ds(k * 16, 16)] = jnp.zeros((16,), jnp.float32)
            return carry

        lax.fori_loop(0, EB, _zfill, 0)
        r0 = s * ROWS_PER_TILE
        nz = ROWS_PER_TILE // EB
        for u in range(nz):
            pltpu.async_copy(
                rows_v.at[u % nbuf], y_sh.at[pl.ds(r0 + u * EB, EB)], semg[0]
            )
        pltpu.sync_copy(src_hbm.at[wid], si_v)
        pltpu.sync_copy(dst_hbm.at[wid], di_v)
        for u in range(nz):
            pltpu.make_async_copy(
                rows_v.at[0], y_sh.at[pl.ds(r0, EB)], semg[0]
            ).wait()
        plsc.subcore_barrier()

        def _gather(j, b):
            pltpu.async_copy(g_hbm.at[si_v.at[j]], rows_v.at[b], semg[b])

        def _gwait(b):
            pltpu.make_async_copy(g_hbm.at[si_v.at[0]], rows_v.at[b], semg[b]).wait()

        def _scat(j, b):
            pltpu.async_copy(rows_v.at[b], y_sh.at[di_v.at[j]], sems[b], add=True)

        def _swait(b):
            pltpu.make_async_copy(rows_v.at[b], y_sh.at[di_v.at[0]], sems[b]).wait()

        for b in range(nbuf):
            _gather(b, b)

        def _group(g, carry):
            j0 = g * nbuf
            for b in range(nbuf):
                _gwait(b)
                _scat(j0 + b, b)
            for b in range(nbuf):
                _swait(b)
                _gather(j0 + nbuf + b, b)
            return carry

        lax.fori_loop(0, ngrp, _group, 0)
        j0 = ngrp * nbuf
        for b in range(nbuf):
            _gwait(b)
            _scat(j0 + b, b)
        for t in range(tail):
            j = (ngrp + 1) * nbuf + t
            _swait(0)
            _gather(j, 0)
            _gwait(0)
            _scat(j, 0)
        _swait(0)
        for b in range(1, nbuf):
            _swait(b)
        plsc.subcore_barrier()
        pltpu.sync_copy(
            y_sh.at[pl.ds(r0, ROWS_PER_TILE)],
            out_hbm.at[c, pl.ds(r0, ROWS_PER_TILE)],
        )

    return _spmm


_spmm_128 = _make_spmm(F1, 2)
_spmm_64 = _make_spmm(F2, 8)
_spmm_32 = _make_spmm(F3, 8)


BN = 400


def _tc_first_body(x_ref, w_ref, d0_ref, d1_ref, g_ref, dinv_ref):
    deg = d0_ref[...] + d1_ref[...] + 1.0
    dinv = lax.rsqrt(deg)
    h = jnp.dot(x_ref[...], w_ref[...], preferred_element_type=jnp.float32)
    g_ref[...] = h * dinv
    dinv_ref[...] = dinv


def _tc_mid_body(s0_ref, s1_ref, g_ref, dinv_ref, b_ref, w_ref, out_ref):
    dinv = dinv_ref[...]
    agg = s0_ref[0] + s1_ref[0] + g_ref[...]
    h = jnp.maximum(agg * dinv + b_ref[...], 0.0)
    out_ref[...] = jnp.dot(h, w_ref[...], preferred_element_type=jnp.float32) * dinv


def _tc_last_body(s0_ref, s1_ref, g_ref, dinv_ref, b_ref, out_ref):
    agg = s0_ref[0] + s1_ref[0] + g_ref[...]
    out_ref[...] = agg * dinv_ref[...] + b_ref[...]


def _row_spec(d):
    return pl.BlockSpec((BN, d), lambda i: (i, 0))


def _plane_spec(p, d):
    return pl.BlockSpec((1, BN, d), lambda i, _p=p: (_p, i, 0))


def _full_spec(r, c):
    return pl.BlockSpec((r, c), lambda i: (0, 0))


def _tc_first(x, w, d0, d1, dw):
    return pl.pallas_call(
        _tc_first_body,
        grid=(N // BN,),
        in_specs=[_row_spec(D_IN), _full_spec(D_IN, dw), _row_spec(1), _row_spec(1)],
        out_specs=[_row_spec(dw), _row_spec(1)],
        out_shape=[
            jax.ShapeDtypeStruct((N, dw), jnp.float32),
            jax.ShapeDtypeStruct((N, 1), jnp.float32),
        ],
    )(x, w, d0, d1)


def _tc_mid(sp, g, dinv, b, w, din, dout):
    return pl.pallas_call(
        _tc_mid_body,
        grid=(N // BN,),
        in_specs=[
            _plane_spec(0, din),
            _plane_spec(1, din),
            _row_spec(din),
            _row_spec(1),
            _full_spec(1, din),
            _full_spec(din, dout),
        ],
        out_specs=_row_spec(dout),
        out_shape=jax.ShapeDtypeStruct((N, dout), jnp.float32),
    )(sp, sp, g, dinv, b, w)


def _tc_last(sp, g, dinv, b, d):
    return pl.pallas_call(
        _tc_last_body,
        grid=(N // BN,),
        in_specs=[
            _plane_spec(0, d),
            _plane_spec(1, d),
            _row_spec(d),
            _row_spec(1),
            _full_spec(1, d),
        ],
        out_specs=_row_spec(d),
        out_shape=jax.ShapeDtypeStruct((N, d), jnp.float32),
    )(sp, sp, g, dinv, b)


def kernel(x, edge_index, W1, b1, W2, b2, W3, b3):
    srcr = edge_index[0].reshape(NW, NB, EB)
    dstr = edge_index[1].reshape(NW, NB, EB)

    degp = _deg_kernel(dstr)
    d0 = degp[0, :N, None]
    d1 = degp[1, :N, None]

    g1, dinv = _tc_first(x, W1, d0, d1, F1)
    s1 = _spmm_128(srcr, dstr, g1)
    g2 = _tc_mid(s1, g1, dinv, b1[None, :], W2, F1, F2)
    s2 = _spmm_64(srcr, dstr, g2)
    g3 = _tc_mid(s2, g2, dinv, b2[None, :], W3, F2, F3)
    s3 = _spmm_32(srcr, dstr, g3)
    return _tc_last(s3, g3, dinv, b3[None, :], F3)


# speedup vs baseline: 30.0811x; 1.0143x over previous
import functools

import jax
import jax.numpy as jnp
from jax import lax
from jax.experimental import pallas as pl
from jax.experimental.pallas import tpu as pltpu
from jax.experimental.pallas import tpu_sc as plsc

N = 10000
E = 320000
D_IN = 128
F1 = 128
F2 = 64
F3 = 32

NC = 2
NS = 16
NW = NC * NS
EB = 80
E_PER_TILE = E // NW
NB = E_PER_TILE // EB
N_PAD = 10240
ROWS_PER_TILE = N_PAD // NS
ZROWS = 128
DEG_PAD = 10240
DEG_PER_TILE = DEG_PAD // NS

_sc_mesh = plsc.VectorSubcoreMesh(
    core_axis_name="c", subcore_axis_name="s", num_cores=NC, num_subcores=NS
)


NBUF = 4
NGRP = NB // NBUF - 1


@functools.partial(
    pl.kernel,
    out_type=jax.ShapeDtypeStruct((NC, DEG_PAD), jnp.float32),
    mesh=_sc_mesh,
    scratch_types=[
        pltpu.VMEM((NB, EB), jnp.int32),
        pltpu.VMEM((EB,), jnp.float32),
        pltpu.VMEM((DEG_PER_TILE,), jnp.float32),
        pltpu.VMEM_SHARED((DEG_PAD,), jnp.float32),
        pltpu.SemaphoreType.DMA,
        pltpu.SemaphoreType.DMA,
        pltpu.SemaphoreType.DMA,
        pltpu.SemaphoreType.DMA,
    ],
)
def _deg_kernel(dst_hbm, out_hbm, di_v, ones_v, zeros_v, deg_sh, s0, s1, s2, s3):
    c = lax.axis_index("c")
    s = lax.axis_index("s")
    wid = s * NC + c
    sems = (s0, s1, s2, s3)
    for k in range(EB // 16):
        ones_v[pl.ds(k * 16, 16)] = jnp.ones((16,), jnp.float32)

    def _zfill(i, carry):
        zeros_v[pl.ds(i * 16, 16)] = jnp.zeros((16,), jnp.float32)
        return carry

    lax.fori_loop(0, DEG_PER_TILE // 16, _zfill, 0)
    pltpu.sync_copy(zeros_v, deg_sh.at[pl.ds(s * DEG_PER_TILE, DEG_PER_TILE)])
    pltpu.sync_copy(dst_hbm.at[wid], di_v)
    plsc.subcore_barrier()

    def _scat(j, sem):
        pltpu.async_copy(ones_v, deg_sh.at[di_v.at[j]], sem, add=True)

    def _wait(sem):
        pltpu.make_async_copy(ones_v, deg_sh.at[di_v.at[0]], sem).wait()

    for b in range(NBUF):
        _scat(b, sems[b])

    def _group(g, carry):
        j0 = g * NBUF
        for b in range(NBUF):
            _wait(sems[b])
            _scat(j0 + NBUF + b, sems[b])
        return carry

    lax.fori_loop(0, NGRP, _group, 0)
    for b in range(NBUF):
        _wait(sems[b])
    pltpu.sync_copy(ones_v, deg_sh.at[di_v.at[NB - 1]], add=True)
    plsc.subcore_barrier()
    pltpu.sync_copy(
        deg_sh.at[pl.ds(s * DEG_PER_TILE, DEG_PER_TILE)],
        out_hbm.at[c, pl.ds(s * DEG_PER_TILE, DEG_PER_TILE)],
    )


E_PER_SUB = E // NS
NBC = E_PER_SUB // EB


def _make_spmm(d, nbuf):
    half = d // 2
    ngrp = NBC // nbuf - 1
    tail = NBC - (ngrp + 1) * nbuf

    @functools.partial(
        pl.kernel,
        out_type=jax.ShapeDtypeStruct((2, N_PAD, half), jnp.float32),
        mesh=_sc_mesh,
        scratch_types=[
            pltpu.VMEM((NBC, EB), jnp.int32),
            pltpu.VMEM((NBC, EB), jnp.int32),
            pltpu.VMEM((nbuf, EB, half), jnp.float32),
            pltpu.VMEM_SHARED((N_PAD, half), jnp.float32),
            [pltpu.SemaphoreType.DMA] * nbuf,
            [pltpu.SemaphoreType.DMA] * nbuf,
        ],
        compiler_params=pltpu.CompilerParams(use_tc_tiling_on_sc=False),
    )
    def _spmm(src_hbm, dst_hbm, glo_hbm, ghi_hbm, out_hbm, si_v, di_v, rows_v,
              y_sh, semg, sems):
        c = lax.axis_index("c")
        s = lax.axis_index("s")

        def _zfill(i, carry):
            for b in range(nbuf):
                for k in range(half // 16):
                    rows_v[b, i, pl.ds(k * 16, 16)] = jnp.zeros((16,), jnp.float32)
            return carry

        lax.fori_loop(0, EB, _zfill, 0)
        r0 = s * ROWS_PER_TILE
        nz = ROWS_PER_TILE // EB
        for u in range(nz):
            pltpu.async_copy(
                rows_v.at[u % nbuf], y_sh.at[pl.ds(r0 + u * EB, EB)], semg[0]
            )
        pltpu.sync_copy(src_hbm.at[s], si_v)
        pltpu.sync_copy(dst_hbm.at[s], di_v)
        for u in range(nz):
            pltpu.make_async_copy(
                rows_v.at[0], y_sh.at[pl.ds(r0, EB)], semg[0]
            ).wait()
        plsc.subcore_barrier()

        def _gather(j, b):
            @pl.when(c == 0)
            def _():
                pltpu.async_copy(glo_hbm.at[si_v.at[j]], rows_v.at[b], semg[b])

            @pl.when(c == 1)
            def _():
                pltpu.async_copy(ghi_hbm.at[si_v.at[j]], rows_v.at[b], semg[b])

        def _gwait(b):
            pltpu.make_async_copy(glo_hbm.at[si_v.at[0]], rows_v.at[b], semg[b]).wait()

        def _scat(j, b):
            pltpu.async_copy(rows_v.at[b], y_sh.at[di_v.at[j]], sems[b], add=True)

        def _swait(b):
            pltpu.make_async_copy(rows_v.at[b], y_sh.at[di_v.at[0]], sems[b]).wait()

        for b in range(nbuf):
            _gather(b, b)

        def _group(g, carry):
            j0 = g * nbuf
            for b in range(nbuf):
                _gwait(b)
                _scat(j0 + b, b)
            for b in range(nbuf):
                _swait(b)
                _gather(j0 + nbuf + b, b)
            return carry

        lax.fori_loop(0, ngrp, _group, 0)
        j0 = ngrp * nbuf
        for b in range(nbuf):
            _gwait(b)
            _scat(j0 + b, b)
        for t in range(tail):
            j = (ngrp + 1) * nbuf + t
            _swait(0)
            _gather(j, 0)
            _gwait(0)
            _scat(j, 0)
        _swait(0)
        for b in range(1, nbuf):
            _swait(b)
        plsc.subcore_barrier()
        pltpu.sync_copy(
            y_sh.at[pl.ds(r0, ROWS_PER_TILE)],
            out_hbm.at[c, pl.ds(r0, ROWS_PER_TILE)],
        )

    return _spmm


_spmm_128 = _make_spmm(F1, 8)
_spmm_64 = _make_spmm(F2, 8)
_spmm_32 = _make_spmm(F3, 8)


BN = 400


def _tc_first_body(x_ref, w_ref, d0_ref, d1_ref, glo_ref, ghi_ref, dinv_ref):
    deg = d0_ref[...] + d1_ref[...] + 1.0
    dinv = lax.rsqrt(deg)
    h = jnp.dot(x_ref[...], w_ref[...], preferred_element_type=jnp.float32)
    g = h * dinv
    half = h.shape[1] // 2
    glo_ref[...] = g[:, :half]
    ghi_ref[...] = g[:, half:]
    dinv_ref[...] = dinv


def _tc_mid_body(s0_ref, s1_ref, glo_ref, ghi_ref, dinv_ref, b_ref, w_ref,
                 olo_ref, ohi_ref):
    dinv = dinv_ref[...]
    agg = jnp.concatenate(
        [s0_ref[0] + glo_ref[...], s1_ref[0] + ghi_ref[...]], axis=1
    )
    h = jnp.maximum(agg * dinv + b_ref[...], 0.0)
    res = jnp.dot(h, w_ref[...], preferred_element_type=jnp.float32) * dinv
    half = res.shape[1] // 2
    olo_ref[...] = res[:, :half]
    ohi_ref[...] = res[:, half:]


def _tc_last_body(s0_ref, s1_ref, glo_ref, ghi_ref, dinv_ref, b_ref, out_ref):
    agg = jnp.concatenate(
        [s0_ref[0] + glo_ref[...], s1_ref[0] + ghi_ref[...]], axis=1
    )
    out_ref[...] = agg * dinv_ref[...] + b_ref[...]


def _row_spec(d):
    return pl.BlockSpec((BN, d), lambda i: (i, 0))


def _plane_spec(p, d):
    return pl.BlockSpec((1, BN, d), lambda i, _p=p: (_p, i, 0))


def _full_spec(r, c):
    return pl.BlockSpec((r, c), lambda i: (0, 0))


def _tc_first(x, w, d0, d1, dw):
    return pl.pallas_call(
        _tc_first_body,
        grid=(N // BN,),
        in_specs=[_row_spec(D_IN), _full_spec(D_IN, dw), _row_spec(1), _row_spec(1)],
        out_specs=[_row_spec(dw // 2), _row_spec(dw // 2), _row_spec(1)],
        out_shape=[
            jax.ShapeDtypeStruct((N, dw // 2), jnp.float32),
            jax.ShapeDtypeStruct((N, dw // 2), jnp.float32),
            jax.ShapeDtypeStruct((N, 1), jnp.float32),
        ],
    )(x, w, d0, d1)


def _tc_mid(sp, glo, ghi, dinv, b, w, din, dout):
    return pl.pallas_call(
        _tc_mid_body,
        grid=(N // BN,),
        in_specs=[
            _plane_spec(0, din // 2),
            _plane_spec(1, din // 2),
            _row_spec(din // 2),
            _row_spec(din // 2),
            _row_spec(1),
            _full_spec(1, din),
            _full_spec(din, dout),
        ],
        out_specs=[_row_spec(dout // 2), _row_spec(dout // 2)],
        out_shape=[
            jax.ShapeDtypeStruct((N, dout // 2), jnp.float32),
            jax.ShapeDtypeStruct((N, dout // 2), jnp.float32),
        ],
    )(sp, sp, glo, ghi, dinv, b, w)


def _tc_last(sp, glo, ghi, dinv, b, d):
    return pl.pallas_call(
        _tc_last_body,
        grid=(N // BN,),
        in_specs=[
            _plane_spec(0, d // 2),
            _plane_spec(1, d // 2),
            _row_spec(d // 2),
            _row_spec(d // 2),
            _row_spec(1),
            _full_spec(1, d),
        ],
        out_specs=_row_spec(d),
        out_shape=jax.ShapeDtypeStruct((N, d), jnp.float32),
    )(sp, sp, glo, ghi, dinv, b)


def kernel(x, edge_index, W1, b1, W2, b2, W3, b3):
    src = edge_index[0]
    dst = edge_index[1]
    dst32 = dst.reshape(NW, NB, EB)
    src16 = src.reshape(NS, NBC, EB)
    dst16 = dst.reshape(NS, NBC, EB)

    degp = _deg_kernel(dst32)
    d0 = degp[0, :N, None]
    d1 = degp[1, :N, None]

    g1lo, g1hi, dinv = _tc_first(x, W1, d0, d1, F1)
    s1 = _spmm_128(src16, dst16, g1lo, g1hi)
    g2lo, g2hi = _tc_mid(s1, g1lo, g1hi, dinv, b1[None, :], W2, F1, F2)
    s2 = _spmm_64(src16, dst16, g2lo, g2hi)
    g3lo, g3hi = _tc_mid(s2, g2lo, g2hi, dinv, b2[None, :], W3, F2, F3)
    s3 = _spmm_32(src16, dst16, g3lo, g3hi)
    return _tc_last(s3, g3lo, g3hi, dinv, b3[None, :], F3)


# speedup vs baseline: 32.2316x; 1.0715x over previous
import functools

import jax
import jax.numpy as jnp
from jax import lax
from jax.experimental import pallas as pl
from jax.experimental.pallas import tpu as pltpu
from jax.experimental.pallas import tpu_sc as plsc

N = 10000
E = 320000
D_IN = 128
F1 = 128
F2 = 64
F3 = 32

NC = 2
NS = 16
NW = NC * NS
EB = 80
E_PER_TILE = E // NW
NB = E_PER_TILE // EB
N_PAD = 10240
ROWS_PER_TILE = N_PAD // NS
ZROWS = 128
DEG_PAD = 10240
DEG_PER_TILE = DEG_PAD // NS

_sc_mesh = plsc.VectorSubcoreMesh(
    core_axis_name="c", subcore_axis_name="s", num_cores=NC, num_subcores=NS
)


NBUF = 4
NGRP = NB // NBUF - 1


@functools.partial(
    pl.kernel,
    out_type=jax.ShapeDtypeStruct((NC, DEG_PAD), jnp.float32),
    mesh=_sc_mesh,
    scratch_types=[
        pltpu.VMEM((NB, EB), jnp.int32),
        pltpu.VMEM((EB,), jnp.float32),
        pltpu.VMEM((DEG_PER_TILE,), jnp.float32),
        pltpu.VMEM_SHARED((DEG_PAD,), jnp.float32),
        pltpu.SemaphoreType.DMA,
        pltpu.SemaphoreType.DMA,
        pltpu.SemaphoreType.DMA,
        pltpu.SemaphoreType.DMA,
    ],
)
def _deg_kernel(dst_hbm, out_hbm, di_v, ones_v, zeros_v, deg_sh, s0, s1, s2, s3):
    c = lax.axis_index("c")
    s = lax.axis_index("s")
    wid = s * NC + c
    sems = (s0, s1, s2, s3)
    for k in range(EB // 16):
        ones_v[pl.ds(k * 16, 16)] = jnp.ones((16,), jnp.float32)

    def _zfill(i, carry):
        zeros_v[pl.ds(i * 16, 16)] = jnp.zeros((16,), jnp.float32)
        return carry

    lax.fori_loop(0, DEG_PER_TILE // 16, _zfill, 0)
    pltpu.sync_copy(zeros_v, deg_sh.at[pl.ds(s * DEG_PER_TILE, DEG_PER_TILE)])
    pltpu.sync_copy(dst_hbm.at[wid], di_v)
    plsc.subcore_barrier()

    def _scat(j, sem):
        pltpu.async_copy(ones_v, deg_sh.at[di_v.at[j]], sem, add=True)

    def _wait(sem):
        pltpu.make_async_copy(ones_v, deg_sh.at[di_v.at[0]], sem).wait()

    for b in range(NBUF):
        _scat(b, sems[b])

    def _group(g, carry):
        j0 = g * NBUF
        for b in range(NBUF):
            _wait(sems[b])
            _scat(j0 + NBUF + b, sems[b])
        return carry

    lax.fori_loop(0, NGRP, _group, 0)
    for b in range(NBUF):
        _wait(sems[b])
    pltpu.sync_copy(ones_v, deg_sh.at[di_v.at[NB - 1]], add=True)
    plsc.subcore_barrier()
    pltpu.sync_copy(
        deg_sh.at[pl.ds(s * DEG_PER_TILE, DEG_PER_TILE)],
        out_hbm.at[c, pl.ds(s * DEG_PER_TILE, DEG_PER_TILE)],
    )


E_PER_SUB = E // NS
NBC = E_PER_SUB // EB


def _make_spmm(d, nbuf):
    half = d // 2
    ngrp = NBC // nbuf - 1
    tail = NBC - (ngrp + 1) * nbuf

    @functools.partial(
        pl.kernel,
        out_type=jax.ShapeDtypeStruct((2, N_PAD, half), jnp.float32),
        mesh=_sc_mesh,
        scratch_types=[
            pltpu.VMEM((NBC, EB), jnp.int32),
            pltpu.VMEM((NBC, EB), jnp.int32),
            pltpu.VMEM((nbuf, EB, half), jnp.float32),
            pltpu.VMEM_SHARED((N_PAD, half), jnp.float32),
            [pltpu.SemaphoreType.DMA] * nbuf,
            [pltpu.SemaphoreType.DMA] * nbuf,
        ],
        compiler_params=pltpu.CompilerParams(use_tc_tiling_on_sc=False),
    )
    def _spmm(src_hbm, dst_hbm, glo_hbm, ghi_hbm, out_hbm, si_v, di_v, rows_v,
              y_sh, semg, sems):
        c = lax.axis_index("c")
        s = lax.axis_index("s")

        pltpu.async_copy(src_hbm.at[s], si_v, sems[0])
        pltpu.async_copy(dst_hbm.at[s], di_v, sems[1])

        def _zfill(i, carry):
            for b in range(nbuf):
                for k in range(half // 16):
                    rows_v[b, i, pl.ds(k * 16, 16)] = jnp.zeros((16,), jnp.float32)
            return carry

        lax.fori_loop(0, EB, _zfill, 0)
        r0 = s * ROWS_PER_TILE
        nz = ROWS_PER_TILE // EB
        for u in range(nz):
            pltpu.async_copy(
                rows_v.at[u % nbuf], y_sh.at[pl.ds(r0 + u * EB, EB)], semg[0]
            )
        for u in range(nz):
            pltpu.make_async_copy(
                rows_v.at[0], y_sh.at[pl.ds(r0, EB)], semg[0]
            ).wait()
        pltpu.make_async_copy(src_hbm.at[s], si_v, sems[0]).wait()
        pltpu.make_async_copy(dst_hbm.at[s], di_v, sems[1]).wait()

        def _gather(j, b):
            @pl.when(c == 0)
            def _():
                pltpu.async_copy(glo_hbm.at[si_v.at[j]], rows_v.at[b], semg[b])

            @pl.when(c == 1)
            def _():
                pltpu.async_copy(ghi_hbm.at[si_v.at[j]], rows_v.at[b], semg[b])

        def _gwait(b):
            pltpu.make_async_copy(glo_hbm.at[si_v.at[0]], rows_v.at[b], semg[b]).wait()

        def _scat(j, b):
            pltpu.async_copy(rows_v.at[b], y_sh.at[di_v.at[j]], sems[b], add=True)

        def _swait(b):
            pltpu.make_async_copy(rows_v.at[b], y_sh.at[di_v.at[0]], sems[b]).wait()

        for b in range(nbuf):
            _gather(b, b)
        plsc.subcore_barrier()

        def _group(g, carry):
            j0 = g * nbuf
            for b in range(nbuf):
                _gwait(b)
                _scat(j0 + b, b)
            for b in range(nbuf):
                _swait(b)
                _gather(j0 + nbuf + b, b)
            return carry

        lax.fori_loop(0, ngrp, _group, 0)
        j0 = ngrp * nbuf
        for b in range(nbuf):
            _gwait(b)
            _scat(j0 + b, b)
        for t in range(tail):
            j = (ngrp + 1) * nbuf + t
            _swait(0)
            _gather(j, 0)
            _gwait(0)
            _scat(j, 0)
        _swait(0)
        for b in range(1, nbuf):
            _swait(b)
        plsc.subcore_barrier()
        pltpu.sync_copy(
            y_sh.at[pl.ds(r0, ROWS_PER_TILE)],
            out_hbm.at[c, pl.ds(r0, ROWS_PER_TILE)],
        )

    return _spmm


def _make_spmm_es(d, nbuf):
    ngrp = NB // nbuf - 1
    tail = NB - (ngrp + 1) * nbuf

    @functools.partial(
        pl.kernel,
        out_type=jax.ShapeDtypeStruct((NC, N_PAD, d), jnp.float32),
        mesh=_sc_mesh,
        scratch_types=[
            pltpu.VMEM((NB, EB), jnp.int32),
            pltpu.VMEM((NB, EB), jnp.int32),
            pltpu.VMEM((nbuf, EB, d), jnp.float32),
            pltpu.VMEM_SHARED((N_PAD, d), jnp.float32),
            [pltpu.SemaphoreType.DMA] * nbuf,
            [pltpu.SemaphoreType.DMA] * nbuf,
        ],
        compiler_params=pltpu.CompilerParams(use_tc_tiling_on_sc=False),
    )
    def _spmm(src_hbm, dst_hbm, g_hbm, out_hbm, si_v, di_v, rows_v, y_sh,
              semg, sems):
        c = lax.axis_index("c")
        s = lax.axis_index("s")
        wid = s * NC + c
        pltpu.async_copy(src_hbm.at[wid], si_v, sems[0])
        pltpu.async_copy(dst_hbm.at[wid], di_v, sems[1])

        def _zfill(i, carry):
            for b in range(nbuf):
                for k in range(d // 16):
                    rows_v[b, i, pl.ds(k * 16, 16)] = jnp.zeros((16,), jnp.float32)
            return carry

        lax.fori_loop(0, EB, _zfill, 0)
        r0 = s * ROWS_PER_TILE
        nz = ROWS_PER_TILE // EB
        for u in range(nz):
            pltpu.async_copy(
                rows_v.at[u % nbuf], y_sh.at[pl.ds(r0 + u * EB, EB)], semg[0]
            )
        for u in range(nz):
            pltpu.make_async_copy(
                rows_v.at[0], y_sh.at[pl.ds(r0, EB)], semg[0]
            ).wait()
        pltpu.make_async_copy(src_hbm.at[wid], si_v, sems[0]).wait()
        pltpu.make_async_copy(dst_hbm.at[wid], di_v, sems[1]).wait()

        def _gather(j, b):
            pltpu.async_copy(g_hbm.at[si_v.at[j]], rows_v.at[b], semg[b])

        def _gwait(b):
            pltpu.make_async_copy(g_hbm.at[si_v.at[0]], rows_v.at[b], semg[b]).wait()

        def _scat(j, b):
            pltpu.async_copy(rows_v.at[b], y_sh.at[di_v.at[j]], sems[b], add=True)

        def _swait(b):
            pltpu.make_async_copy(rows_v.at[b], y_sh.at[di_v.at[0]], sems[b]).wait()

        for b in range(nbuf):
            _gather(b, b)
        plsc.subcore_barrier()

        def _group(g, carry):
            j0 = g * nbuf
            for b in range(nbuf):
                _gwait(b)
                _scat(j0 + b, b)
            for b in range(nbuf):
                _swait(b)
                _gather(j0 + nbuf + b, b)
            return carry

        lax.fori_loop(0, ngrp, _group, 0)
        j0 = ngrp * nbuf
        for b in range(nbuf):
            _gwait(b)
            _scat(j0 + b, b)
        for t in range(tail):
            j = (ngrp + 1) * nbuf + t
            _swait(0)
            _gather(j, 0)
            _gwait(0)
            _scat(j, 0)
        _swait(0)
        for b in range(1, nbuf):
            _swait(b)
        plsc.subcore_barrier()
        pltpu.sync_copy(
            y_sh.at[pl.ds(r0, ROWS_PER_TILE)],
            out_hbm.at[c, pl.ds(r0, ROWS_PER_TILE)],
        )

    return _spmm


_spmm_128 = _make_spmm(F1, 8)
_spmm_64 = _make_spmm_es(F2, 8)
_spmm_32 = _make_spmm_es(F3, 8)


BN = 400


def _tc_first_body(x_ref, w_ref, d0_ref, d1_ref, glo_ref, ghi_ref, dinv_ref):
    deg = d0_ref[...] + d1_ref[...] + 1.0
    dinv = lax.rsqrt(deg)
    h = jnp.dot(x_ref[...], w_ref[...], preferred_element_type=jnp.float32)
    g = h * dinv
    half = h.shape[1] // 2
    glo_ref[...] = g[:, :half]
    ghi_ref[...] = g[:, half:]
    dinv_ref[...] = dinv


def _tc_mid_cs_body(s0_ref, s1_ref, glo_ref, ghi_ref, dinv_ref, b_ref, w_ref,
                    out_ref):
    dinv = dinv_ref[...]
    agg = jnp.concatenate(
        [s0_ref[0] + glo_ref[...], s1_ref[0] + ghi_ref[...]], axis=1
    )
    h = jnp.maximum(agg * dinv + b_ref[...], 0.0)
    out_ref[...] = jnp.dot(h, w_ref[...], preferred_element_type=jnp.float32) * dinv


def _tc_mid_es_body(s0_ref, s1_ref, g_ref, dinv_ref, b_ref, w_ref, out_ref):
    dinv = dinv_ref[...]
    agg = s0_ref[0] + s1_ref[0] + g_ref[...]
    h = jnp.maximum(agg * dinv + b_ref[...], 0.0)
    out_ref[...] = jnp.dot(h, w_ref[...], preferred_element_type=jnp.float32) * dinv


def _tc_last_body(s0_ref, s1_ref, g_ref, dinv_ref, b_ref, out_ref):
    agg = s0_ref[0] + s1_ref[0] + g_ref[...]
    out_ref[...] = agg * dinv_ref[...] + b_ref[...]


def _row_spec(d):
    return pl.BlockSpec((BN, d), lambda i: (i, 0))


def _plane_spec(p, d):
    return pl.BlockSpec((1, BN, d), lambda i, _p=p: (_p, i, 0))


def _full_spec(r, c):
    return pl.BlockSpec((r, c), lambda i: (0, 0))


def _tc_first(x, w, d0, d1, dw):
    return pl.pallas_call(
        _tc_first_body,
        grid=(N // BN,),
        in_specs=[_row_spec(D_IN), _full_spec(D_IN, dw), _row_spec(1), _row_spec(1)],
        out_specs=[_row_spec(dw // 2), _row_spec(dw // 2), _row_spec(1)],
        out_shape=[
            jax.ShapeDtypeStruct((N, dw // 2), jnp.float32),
            jax.ShapeDtypeStruct((N, dw // 2), jnp.float32),
            jax.ShapeDtypeStruct((N, 1), jnp.float32),
        ],
    )(x, w, d0, d1)


def _tc_mid_cs(sp, glo, ghi, dinv, b, w, din, dout):
    return pl.pallas_call(
        _tc_mid_cs_body,
        grid=(N // BN,),
        in_specs=[
            _plane_spec(0, din // 2),
            _plane_spec(1, din // 2),
            _row_spec(din // 2),
            _row_spec(din // 2),
            _row_spec(1),
            _full_spec(1, din),
            _full_spec(din, dout),
        ],
        out_specs=_row_spec(dout),
        out_shape=jax.ShapeDtypeStruct((N, dout), jnp.float32),
    )(sp, sp, glo, ghi, dinv, b, w)


def _tc_mid_es(sp, g, dinv, b, w, din, dout):
    return pl.pallas_call(
        _tc_mid_es_body,
        grid=(N // BN,),
        in_specs=[
            _plane_spec(0, din),
            _plane_spec(1, din),
            _row_spec(din),
            _row_spec(1),
            _full_spec(1, din),
            _full_spec(din, dout),
        ],
        out_specs=_row_spec(dout),
        out_shape=jax.ShapeDtypeStruct((N, dout), jnp.float32),
    )(sp, sp, g, dinv, b, w)


def _tc_last(sp, g, dinv, b, d):
    return pl.pallas_call(
        _tc_last_body,
        grid=(N // BN,),
        in_specs=[
            _plane_spec(0, d),
            _plane_spec(1, d),
            _row_spec(d),
            _row_spec(1),
            _full_spec(1, d),
        ],
        out_specs=_row_spec(d),
        out_shape=jax.ShapeDtypeStruct((N, d), jnp.float32),
    )(sp, sp, g, dinv, b)


def kernel(x, edge_index, W1, b1, W2, b2, W3, b3):
    src = edge_index[0]
    dst = edge_index[1]
    src32 = src.reshape(NW, NB, EB)
    dst32 = dst.reshape(NW, NB, EB)
    src16 = src.reshape(NS, NBC, EB)
    dst16 = dst.reshape(NS, NBC, EB)

    degp = _deg_kernel(dst32)
    d0 = degp[0, :N, None]
    d1 = degp[1, :N, None]

    g1lo, g1hi, dinv = _tc_first(x, W1, d0, d1, F1)
    s1 = _spmm_128(src16, dst16, g1lo, g1hi)
    g2 = _tc_mid_cs(s1, g1lo, g1hi, dinv, b1[None, :], W2, F1, F2)
    s2 = _spmm_64(src32, dst32, g2)
    g3 = _tc_mid_es(s2, g2, dinv, b2[None, :], W3, F2, F3)
    s3 = _spmm_32(src32, dst32, g3)
    return _tc_last(s3, g3, dinv, b3[None, :], F3)
